# baseline (speedup 1.0000x reference)
import jax
import jax.numpy as jnp
from jax import lax
from jax.experimental import pallas as pl
from jax.experimental.pallas import tpu as pltpu
from jax.experimental.pallas import tpu_sc as plsc

_B, _Q, _C = 64, 2048, 256
_NC, _NS, _L = 2, 16, 16
_NW = _NC * _NS
_QS = 768
_QT = _Q - _QS
_BPW = 2
_RPW = _BPW * _QS
_CHUNK = 128
_KQ = _QS // _CHUNK
_NCHUNK = _BPW * _KQ
_GROUPS = _CHUNK // _L
_BLK = 8
_MAIN_C = 240


def _sc_body(logits_hbm, boxes_hbm, labels_hbm, boxes_out_hbm, counts_hbm,
             lbuf0, lbuf1, bbuf, lab_st, box_st, vscr, sem_b, sem0, sem1):
    cid = lax.axis_index("c")
    sid = lax.axis_index("s")
    wid = sid * _NC + cid
    b0 = wid * _BPW

    lane = lax.iota(jnp.int32, _L)
    lane_d4 = jnp.right_shift(lane, 2)
    box_sel = [lane_d4 + (4 * k) for k in range(4)]
    neg_inf = jnp.full((_L,), -jnp.inf, jnp.float32)

    lbufs = (lbuf0, lbuf1)
    sems = (sem0, sem1)

    def start_chunk_dma(g, buf, sem):
        bb = b0 + g // _KQ
        q0 = (g % _KQ) * _CHUNK
        pltpu.async_copy(logits_hbm.at[bb, pl.ds(q0, _CHUNK), :], buf, sem)

    for i in range(_BPW):
        cp = pltpu.async_copy(
            boxes_hbm.at[b0 + i, pl.ds(0, _QS * 4)],
            bbuf.at[pl.ds(i * _QS * 4, _QS * 4)], sem_b)
    start_chunk_dma(jnp.int32(0), lbuf0, sem0)
    start_chunk_dma(jnp.int32(1), lbuf1, sem1)
    cp.wait()
    cp.wait()

    def chunk_step(g, b, cnt):
        buf = lbufs[b]
        sem = sems[b]
        pltpu.make_async_copy(
            logits_hbm.at[0, pl.ds(0, _CHUNK), :], buf, sem).wait()

        def grp_body(grp, cnt):
            row0 = g * _CHUNK + grp * _L
            rows = lane + grp * _L

            def blk_body(blk, carry):
                best, bblk = carry
                c = blk * _BLK
                vs = []
                col = lane + c
                for j in range(_BLK):
                    if j:
                        col = col + 1
                    vs.append(plsc.load_gather(buf, [rows, col]))
                m01 = jnp.maximum(vs[0], vs[1])
                m23 = jnp.maximum(vs[2], vs[3])
                m45 = jnp.maximum(vs[4], vs[5])
                m67 = jnp.maximum(vs[6], vs[7])
                m = jnp.maximum(jnp.maximum(m01, m23),
                                jnp.maximum(m45, m67))
                gt = m > best
                best = jnp.where(gt, m, best)
                bblk = jnp.where(gt, jnp.full((_L,), c, jnp.int32), bblk)
                return (best, bblk)

            best, bblk = lax.fori_loop(
                0, _MAIN_C // _BLK, blk_body,
                (neg_inf, jnp.zeros((_L,), jnp.int32)))

            col = bblk + lane
            v = plsc.load_gather(buf, [rows, col])
            bcol = col
            found = v == best
            for _ in range(_BLK - 1):
                col = col + 1
                v = plsc.load_gather(buf, [rows, col])
                hit = jnp.logical_and(v == best,
                                      jnp.logical_not(found))
                bcol = jnp.where(hit, col, bcol)
                found = jnp.logical_or(found, hit)

            def tail_body(_, carry):
                best, bcol, col = carry
                col = jnp.bitwise_and(col + 1, _C - 1)
                v = plsc.load_gather(buf, [rows, col])
                gt = v > best
                best = jnp.where(gt, v, best)
                bcol = jnp.where(gt, col, bcol)
                return (best, bcol, col)

            best, bcol, _ = lax.fori_loop(
                _MAIN_C, _C, tail_body,
                (best, bcol, lane + (_MAIN_C - 1)))

            valid = best > 0.0
            lab_st[pl.ds(row0, _L)] = jnp.where(valid, bcol, -1)
            cnt = cnt + jnp.where(valid, 1.0, 0.0)
            vscr[...] = jnp.where(valid, 1.0, 0.0)
            off = row0 * 4
            for k in range(4):
                mv = plsc.load_gather(vscr, [box_sel[k]])
                bx = bbuf[pl.ds(off + k * _L, _L)]
                box_st[pl.ds(off + k * _L, _L)] = jnp.where(
                    mv > 0.0, bx, 0.0)
            return cnt

        cnt = lax.fori_loop(0, _GROUPS, grp_body, cnt)

        @pl.when(g + 2 < _NCHUNK)
        def _():
            start_chunk_dma(g + 2, buf, sem)

        return cnt

    def pair_body(p, cnt):
        g = p * 2
        cnt = chunk_step(g, 0, cnt)
        cnt = chunk_step(g + 1, 1, cnt)
        return cnt

    cnt = lax.fori_loop(0, _NCHUNK // 2, pair_body,
                        jnp.zeros((_L,), jnp.float32))

    vscr[...] = cnt
    pltpu.sync_copy(vscr, counts_hbm.at[wid])
    for i in range(_BPW):
        pltpu.sync_copy(lab_st.at[pl.ds(i * _QS, _QS)], labels_hbm.at[b0 + i])
        pltpu.sync_copy(box_st.at[pl.ds(i * _QS * 4, _QS * 4)],
                        boxes_out_hbm.at[b0 + i])


_TB = 8


def _tc_body(x_ref, bx_ref, lab_ref, bo_ref, cnt_ref):
    i = pl.program_id(0)
    j = pl.program_id(1)
    x = x_ref[...]
    m = jnp.max(x, axis=-1, keepdims=True)
    iota = lax.broadcasted_iota(jnp.int32, (_TB, _CHUNK, _C), 2)
    a = jnp.min(jnp.where(x == m, iota, _C), axis=-1)
    valid = m > 0.0
    lab_ref[...] = jnp.where(jnp.max(x, axis=-1) > 0.0, a, -1)
    bo_ref[...] = jnp.where(valid, bx_ref[...], 0.0)
    c = jnp.sum(jnp.where(valid, 1.0, 0.0))

    @pl.when(jnp.logical_and(i == 0, j == 0))
    def _():
        cnt_ref[...] = jnp.zeros_like(cnt_ref)

    cnt_ref[...] += lax.broadcast(c, (1, 1))


def _finalize_body(cref, tref, oref):
    oref[...] = jnp.maximum(jnp.sum(cref[...]) + tref[...], 1.0)


def kernel(pred_logits, pred_boxes):
    boxes2d = pred_boxes.reshape(_B, _Q * 4)
    mesh = plsc.VectorSubcoreMesh(core_axis_name="c", subcore_axis_name="s")
    labels_sc, boxes2d_sc, counts = pl.kernel(
        _sc_body,
        out_type=(
            jax.ShapeDtypeStruct((_B, _QS), jnp.int32),
            jax.ShapeDtypeStruct((_B, _QS * 4), jnp.float32),
            jax.ShapeDtypeStruct((_NW, _L), jnp.float32),
        ),
        mesh=mesh,
        compiler_params=pltpu.CompilerParams(needs_layout_passes=False),
        scratch_types=[
            pltpu.VMEM((_CHUNK, _C), jnp.float32),
            pltpu.VMEM((_CHUNK, _C), jnp.float32),
            pltpu.VMEM((_RPW * 4,), jnp.float32),
            pltpu.VMEM((_RPW,), jnp.int32),
            pltpu.VMEM((_RPW * 4,), jnp.float32),
            pltpu.VMEM((_L,), jnp.float32),
            pltpu.SemaphoreType.DMA,
            pltpu.SemaphoreType.DMA,
            pltpu.SemaphoreType.DMA,
        ],
    )(pred_logits, boxes2d)
    labels_tc, boxes_tc, cnt_tc = pl.pallas_call(
        _tc_body,
        grid=(_B // _TB, _QT // _CHUNK),
        in_specs=[
            pl.BlockSpec((_TB, _CHUNK, _C), lambda i, j: (i, _KQ + j, 0)),
            pl.BlockSpec((_TB, _CHUNK, 4), lambda i, j: (i, _KQ + j, 0)),
        ],
        out_specs=[
            pl.BlockSpec((_TB, _CHUNK), lambda i, j: (i, j)),
            pl.BlockSpec((_TB, _CHUNK, 4), lambda i, j: (i, j, 0)),
            pl.BlockSpec((1, 1), lambda i, j: (0, 0)),
        ],
        out_shape=[
            jax.ShapeDtypeStruct((_B, _QT), jnp.int32),
            jax.ShapeDtypeStruct((_B, _QT, 4), jnp.float32),
            jax.ShapeDtypeStruct((1, 1), jnp.float32),
        ],
    )(pred_logits, pred_boxes)
    num_boxes = pl.pallas_call(
        _finalize_body,
        out_shape=jax.ShapeDtypeStruct((1, 1), jnp.float32),
    )(counts, cnt_tc)[0, 0]
    labels = jnp.concatenate([labels_sc, labels_tc], axis=1)
    boxes_out = jnp.concatenate(
        [boxes2d_sc.reshape(_B, _QS, 4), boxes_tc], axis=1)
    return labels, boxes_out, num_boxes


# speedup vs baseline: 1.4836x; 1.4836x over previous
import jax
import jax.numpy as jnp
from jax import lax
from jax.experimental import pallas as pl
from jax.experimental.pallas import tpu as pltpu
from jax.experimental.pallas import tpu_sc as plsc

_B, _Q, _C = 64, 2048, 256
_NC, _NS, _L = 2, 16, 16
_NW = _NC * _NS
_QS = 768
_QT = _Q - _QS
_BPW = 2
_RPW = _BPW * _QS
_CHUNK = 128
_KQ = _QS // _CHUNK
_NCHUNK = _BPW * _KQ
_GROUPS = _CHUNK // _L
_BLK = 8
_MAIN_C = 240


def _sc_body(logits_hbm, boxes_hbm, labels_hbm, boxes_out_hbm, counts_hbm,
             lbuf0, lbuf1, bbuf, lab_st, box_st, vscr, sem_b, sem0, sem1):
    cid = lax.axis_index("c")
    sid = lax.axis_index("s")
    wid = sid * _NC + cid
    b0 = wid * _BPW

    lane = lax.iota(jnp.int32, _L)
    lane_d4 = jnp.right_shift(lane, 2)
    box_sel = [lane_d4 + (4 * k) for k in range(4)]
    neg_inf = jnp.full((_L,), -jnp.inf, jnp.float32)

    lbufs = (lbuf0, lbuf1)
    sems = (sem0, sem1)

    def start_chunk_dma(g, buf, sem):
        bb = b0 + g // _KQ
        q0 = (g % _KQ) * _CHUNK
        pltpu.async_copy(logits_hbm.at[bb, pl.ds(q0, _CHUNK), :], buf, sem)

    for i in range(_BPW):
        cp = pltpu.async_copy(
            boxes_hbm.at[b0 + i, pl.ds(0, _QS * 4)],
            bbuf.at[pl.ds(i * _QS * 4, _QS * 4)], sem_b)
    start_chunk_dma(jnp.int32(0), lbuf0, sem0)
    start_chunk_dma(jnp.int32(1), lbuf1, sem1)
    cp.wait()
    cp.wait()

    def chunk_step(g, b, cnt):
        buf = lbufs[b]
        sem = sems[b]
        pltpu.make_async_copy(
            logits_hbm.at[0, pl.ds(0, _CHUNK), :], buf, sem).wait()

        def grp_body(grp, cnt):
            row0 = g * _CHUNK + grp * _L
            rows = lane + grp * _L

            def blk_body(blk, carry):
                best, bblk = carry
                c = blk * _BLK
                vs = []
                col = lane + c
                for j in range(_BLK):
                    if j:
                        col = col + 1
                    vs.append(plsc.load_gather(buf, [rows, col]))
                m01 = jnp.maximum(vs[0], vs[1])
                m23 = jnp.maximum(vs[2], vs[3])
                m45 = jnp.maximum(vs[4], vs[5])
                m67 = jnp.maximum(vs[6], vs[7])
                m = jnp.maximum(jnp.maximum(m01, m23),
                                jnp.maximum(m45, m67))
                gt = m > best
                best = jnp.where(gt, m, best)
                bblk = jnp.where(gt, jnp.full((_L,), c, jnp.int32), bblk)
                return (best, bblk)

            best, bblk = lax.fori_loop(
                0, _MAIN_C // _BLK, blk_body,
                (neg_inf, jnp.zeros((_L,), jnp.int32)))

            col = bblk + lane
            v = plsc.load_gather(buf, [rows, col])
            bcol = col
            found = v == best
            for _ in range(_BLK - 1):
                col = col + 1
                v = plsc.load_gather(buf, [rows, col])
                hit = jnp.logical_and(v == best,
                                      jnp.logical_not(found))
                bcol = jnp.where(hit, col, bcol)
                found = jnp.logical_or(found, hit)

            def tail_body(_, carry):
                best, bcol, col = carry
                col = jnp.bitwise_and(col + 1, _C - 1)
                v = plsc.load_gather(buf, [rows, col])
                gt = v > best
                best = jnp.where(gt, v, best)
                bcol = jnp.where(gt, col, bcol)
                return (best, bcol, col)

            best, bcol, _ = lax.fori_loop(
                _MAIN_C, _C, tail_body,
                (best, bcol, lane + (_MAIN_C - 1)))

            valid = best > 0.0
            lab_st[pl.ds(row0, _L)] = jnp.where(valid, bcol, -1)
            cnt = cnt + jnp.where(valid, 1.0, 0.0)
            vscr[...] = jnp.where(valid, 1.0, 0.0)
            off = row0 * 4
            for k in range(4):
                mv = plsc.load_gather(vscr, [box_sel[k]])
                bx = bbuf[pl.ds(off + k * _L, _L)]
                box_st[pl.ds(off + k * _L, _L)] = jnp.where(
                    mv > 0.0, bx, 0.0)
            return cnt

        cnt = lax.fori_loop(0, _GROUPS, grp_body, cnt)

        @pl.when(g + 2 < _NCHUNK)
        def _():
            start_chunk_dma(g + 2, buf, sem)

        return cnt

    def pair_body(p, cnt):
        g = p * 2
        cnt = chunk_step(g, 0, cnt)
        cnt = chunk_step(g + 1, 1, cnt)
        return cnt

    cnt = lax.fori_loop(0, _NCHUNK // 2, pair_body,
                        jnp.zeros((_L,), jnp.float32))

    vscr[...] = cnt
    pltpu.sync_copy(vscr, counts_hbm.at[wid])
    for i in range(_BPW):
        pltpu.sync_copy(lab_st.at[pl.ds(i * _QS, _QS)], labels_hbm.at[b0 + i])
        pltpu.sync_copy(box_st.at[pl.ds(i * _QS * 4, _QS * 4)],
                        boxes_out_hbm.at[b0 + i])


_TB = 8


def _tc_body(x_ref, lab_ref, cnt_ref):
    i = pl.program_id(0)
    j = pl.program_id(1)
    x = x_ref[...]
    m = jnp.max(x, axis=-1, keepdims=True)
    iota = lax.broadcasted_iota(jnp.int32, (_TB, _CHUNK, _C), 2)
    a = jnp.min(jnp.where(x == m, iota, _C), axis=-1)
    valid = jnp.max(x, axis=-1) > 0.0
    lab_ref[...] = jnp.where(valid, a, -1)
    c = jnp.sum(jnp.where(valid, 1.0, 0.0))

    @pl.when(jnp.logical_and(i == 0, j == 0))
    def _():
        cnt_ref[...] = jnp.zeros_like(cnt_ref)

    cnt_ref[...] += lax.broadcast(c, (1, 1))


def _sc2_body(labels_hbm, boxes_hbm, boxes_out_hbm, lbl_st, bbuf, box_st,
              vscr, sem_l, sem_b):
    cid = lax.axis_index("c")
    sid = lax.axis_index("s")
    wid = sid * _NC + cid
    b0 = wid * _BPW

    lane = lax.iota(jnp.int32, _L)
    lane_d4 = jnp.right_shift(lane, 2)
    box_sel = [lane_d4 + (4 * k) for k in range(4)]

    for i in range(_BPW):
        cpl = pltpu.async_copy(
            labels_hbm.at[b0 + i], lbl_st.at[pl.ds(i * _QT, _QT)], sem_l)
        cpb = pltpu.async_copy(
            boxes_hbm.at[b0 + i, pl.ds(_QS * 4, _QT * 4)],
            bbuf.at[pl.ds(i * _QT * 4, _QT * 4)], sem_b)
    cpl.wait()
    cpl.wait()
    cpb.wait()
    cpb.wait()

    def grp_body(grp, _):
        row0 = grp * _L
        lv = lbl_st[pl.ds(row0, _L)]
        vscr[...] = jnp.where(lv >= 0, 1.0, 0.0)
        off = row0 * 4
        for k in range(4):
            mv = plsc.load_gather(vscr, [box_sel[k]])
            bx = bbuf[pl.ds(off + k * _L, _L)]
            box_st[pl.ds(off + k * _L, _L)] = jnp.where(mv > 0.0, bx, 0.0)
        return 0

    lax.fori_loop(0, (_BPW * _QT) // _L, grp_body, 0)

    for i in range(_BPW):
        pltpu.sync_copy(box_st.at[pl.ds(i * _QT * 4, _QT * 4)],
                        boxes_out_hbm.at[b0 + i])


def _finalize_body(cref, tref, oref):
    oref[...] = jnp.maximum(jnp.sum(cref[...]) + tref[...], 1.0)


def kernel(pred_logits, pred_boxes):
    boxes2d = pred_boxes.reshape(_B, _Q * 4)
    mesh = plsc.VectorSubcoreMesh(core_axis_name="c", subcore_axis_name="s")
    labels_sc, boxes2d_sc, counts = pl.kernel(
        _sc_body,
        out_type=(
            jax.ShapeDtypeStruct((_B, _QS), jnp.int32),
            jax.ShapeDtypeStruct((_B, _QS * 4), jnp.float32),
            jax.ShapeDtypeStruct((_NW, _L), jnp.float32),
        ),
        mesh=mesh,
        compiler_params=pltpu.CompilerParams(needs_layout_passes=False),
        scratch_types=[
            pltpu.VMEM((_CHUNK, _C), jnp.float32),
            pltpu.VMEM((_CHUNK, _C), jnp.float32),
            pltpu.VMEM((_RPW * 4,), jnp.float32),
            pltpu.VMEM((_RPW,), jnp.int32),
            pltpu.VMEM((_RPW * 4,), jnp.float32),
            pltpu.VMEM((_L,), jnp.float32),
            pltpu.SemaphoreType.DMA,
            pltpu.SemaphoreType.DMA,
            pltpu.SemaphoreType.DMA,
        ],
    )(pred_logits, boxes2d)
    labels_tc, cnt_tc = pl.pallas_call(
        _tc_body,
        grid=(_B // _TB, _QT // _CHUNK),
        in_specs=[
            pl.BlockSpec((_TB, _CHUNK, _C), lambda i, j: (i, _KQ + j, 0)),
        ],
        out_specs=[
            pl.BlockSpec((_TB, _CHUNK), lambda i, j: (i, j)),
            pl.BlockSpec((1, 1), lambda i, j: (0, 0)),
        ],
        out_shape=[
            jax.ShapeDtypeStruct((_B, _QT), jnp.int32),
            jax.ShapeDtypeStruct((1, 1), jnp.float32),
        ],
    )(pred_logits)
    boxes2d_tc = pl.kernel(
        _sc2_body,
        out_type=jax.ShapeDtypeStruct((_B, _QT * 4), jnp.float32),
        mesh=mesh,
        compiler_params=pltpu.CompilerParams(needs_layout_passes=False),
        scratch_types=[
            pltpu.VMEM((_BPW * _QT,), jnp.int32),
            pltpu.VMEM((_BPW * _QT * 4,), jnp.float32),
            pltpu.VMEM((_BPW * _QT * 4,), jnp.float32),
            pltpu.VMEM((_L,), jnp.float32),
            pltpu.SemaphoreType.DMA,
            pltpu.SemaphoreType.DMA,
        ],
    )(labels_tc, boxes2d)
    num_boxes = pl.pallas_call(
        _finalize_body,
        out_shape=jax.ShapeDtypeStruct((1, 1), jnp.float32),
    )(counts, cnt_tc)[0, 0]
    labels = jnp.concatenate([labels_sc, labels_tc], axis=1)
    boxes_out = jnp.concatenate(
        [boxes2d_sc.reshape(_B, _QS, 4),
         boxes2d_tc.reshape(_B, _QT, 4)], axis=1)
    return labels, boxes_out, num_boxes


# speedup vs baseline: 1.8681x; 1.2592x over previous
import jax
import jax.numpy as jnp
from jax import lax
from jax.experimental import pallas as pl
from jax.experimental.pallas import tpu as pltpu
from jax.experimental.pallas import tpu_sc as plsc

_B, _Q, _C = 64, 2048, 256
_NC, _NS, _L = 2, 16, 16
_NW = _NC * _NS
_QS = 768
_QT = _Q - _QS
_BPW = 2
_RPW = _BPW * _QS
_CHUNK = 128
_KQ = _QS // _CHUNK
_NCHUNK = _BPW * _KQ
_GROUPS = _CHUNK // _L
_BLK = 8
_MAIN_C = 240


def _sc_body(logits_hbm, boxes_hbm, labels_hbm, boxes_out_hbm, counts_hbm,
             lbuf0, lbuf1, bbuf, lab_st, box_st, vscr, sem_b, sem0, sem1):
    cid = lax.axis_index("c")
    sid = lax.axis_index("s")
    wid = sid * _NC + cid
    b0 = wid * _BPW

    lane = lax.iota(jnp.int32, _L)
    lane_d4 = jnp.right_shift(lane, 2)
    box_sel = [lane_d4 + (4 * k) for k in range(4)]
    neg_inf = jnp.full((_L,), -jnp.inf, jnp.float32)

    lbufs = (lbuf0, lbuf1)
    sems = (sem0, sem1)

    def start_chunk_dma(g, buf, sem):
        bb = b0 + g // _KQ
        q0 = (g % _KQ) * _CHUNK
        pltpu.async_copy(logits_hbm.at[bb, pl.ds(q0, _CHUNK), :], buf, sem)

    for i in range(_BPW):
        cp = pltpu.async_copy(
            boxes_hbm.at[b0 + i, pl.ds(0, _QS * 4)],
            bbuf.at[pl.ds(i * _QS * 4, _QS * 4)], sem_b)
    start_chunk_dma(jnp.int32(0), lbuf0, sem0)
    start_chunk_dma(jnp.int32(1), lbuf1, sem1)
    cp.wait()
    cp.wait()

    def chunk_step(g, b, cnt):
        buf = lbufs[b]
        sem = sems[b]
        pltpu.make_async_copy(
            logits_hbm.at[0, pl.ds(0, _CHUNK), :], buf, sem).wait()

        def grp_body(grp, cnt):
            row0 = g * _CHUNK + grp * _L
            rows = lane + grp * _L

            def blk_body(blk, carry):
                best, bblk = carry
                c = blk * _BLK
                vs = []
                col = lane + c
                for j in range(_BLK):
                    if j:
                        col = col + 1
                    vs.append(plsc.load_gather(buf, [rows, col]))
                m01 = jnp.maximum(vs[0], vs[1])
                m23 = jnp.maximum(vs[2], vs[3])
                m45 = jnp.maximum(vs[4], vs[5])
                m67 = jnp.maximum(vs[6], vs[7])
                m = jnp.maximum(jnp.maximum(m01, m23),
                                jnp.maximum(m45, m67))
                gt = m > best
                best = jnp.where(gt, m, best)
                bblk = jnp.where(gt, jnp.full((_L,), c, jnp.int32), bblk)
                return (best, bblk)

            best, bblk = lax.fori_loop(
                0, _MAIN_C // _BLK, blk_body,
                (neg_inf, jnp.zeros((_L,), jnp.int32)))

            col = bblk + lane
            v = plsc.load_gather(buf, [rows, col])
            bcol = col
            found = v == best
            for _ in range(_BLK - 1):
                col = col + 1
                v = plsc.load_gather(buf, [rows, col])
                hit = jnp.logical_and(v == best,
                                      jnp.logical_not(found))
                bcol = jnp.where(hit, col, bcol)
                found = jnp.logical_or(found, hit)

            def tail_body(_, carry):
                best, bcol, col = carry
                col = jnp.bitwise_and(col + 1, _C - 1)
                v = plsc.load_gather(buf, [rows, col])
                gt = v > best
                best = jnp.where(gt, v, best)
                bcol = jnp.where(gt, col, bcol)
                return (best, bcol, col)

            best, bcol, _ = lax.fori_loop(
                _MAIN_C, _C, tail_body,
                (best, bcol, lane + (_MAIN_C - 1)))

            valid = best > 0.0
            lab_st[pl.ds(row0, _L)] = jnp.where(valid, bcol, -1)
            cnt = cnt + jnp.where(valid, 1.0, 0.0)
            vscr[...] = jnp.where(valid, 1.0, 0.0)
            off = row0 * 4
            for k in range(4):
                mv = plsc.load_gather(vscr, [box_sel[k]])
                bx = bbuf[pl.ds(off + k * _L, _L)]
                box_st[pl.ds(off + k * _L, _L)] = jnp.where(
                    mv > 0.0, bx, 0.0)
            return cnt

        cnt = lax.fori_loop(0, _GROUPS, grp_body, cnt)

        @pl.when(g + 2 < _NCHUNK)
        def _():
            start_chunk_dma(g + 2, buf, sem)

        return cnt

    def pair_body(p, cnt):
        g = p * 2
        cnt = chunk_step(g, 0, cnt)
        cnt = chunk_step(g + 1, 1, cnt)
        return cnt

    cnt = lax.fori_loop(0, _NCHUNK // 2, pair_body,
                        jnp.zeros((_L,), jnp.float32))

    vscr[...] = cnt
    pltpu.sync_copy(vscr, counts_hbm.at[wid])
    for i in range(_BPW):
        pltpu.sync_copy(lab_st.at[pl.ds(i * _QS, _QS)], labels_hbm.at[b0 + i])
        pltpu.sync_copy(box_st.at[pl.ds(i * _QS * 4, _QS * 4)],
                        boxes_out_hbm.at[b0 + i])


_TB = 16


def _tc_body(x_ref, lab_ref, cnt_ref):
    i = pl.program_id(0)
    j = pl.program_id(1)
    x = x_ref[...]
    m = jnp.max(x, axis=-1, keepdims=True)
    iota = lax.broadcasted_iota(
        jnp.int32, (_TB, _CHUNK, _C), 2).astype(jnp.float32)
    af = jnp.min(jnp.where(x == m, iota, float(_C)), axis=-1)
    valid = jnp.max(x, axis=-1) > 0.0
    lab_ref[...] = jnp.where(valid, af.astype(jnp.int32), -1)
    c = jnp.sum(jnp.where(valid, 1.0, 0.0))

    @pl.when(jnp.logical_and(i == 0, j == 0))
    def _():
        cnt_ref[...] = jnp.zeros_like(cnt_ref)

    cnt_ref[...] += lax.broadcast(c, (1, 1))


def _sc2_body(labels_hbm, boxes_hbm, boxes_out_hbm, lbl_st, bbuf, box_st,
              vscr, sem_l, sem_b):
    cid = lax.axis_index("c")
    sid = lax.axis_index("s")
    wid = sid * _NC + cid
    b0 = wid * _BPW

    lane = lax.iota(jnp.int32, _L)
    lane_d4 = jnp.right_shift(lane, 2)
    box_sel = [lane_d4 + (4 * k) for k in range(4)]

    for i in range(_BPW):
        cpl = pltpu.async_copy(
            labels_hbm.at[b0 + i], lbl_st.at[pl.ds(i * _QT, _QT)], sem_l)
        cpb = pltpu.async_copy(
            boxes_hbm.at[b0 + i, pl.ds(_QS * 4, _QT * 4)],
            bbuf.at[pl.ds(i * _QT * 4, _QT * 4)], sem_b)
    cpl.wait()
    cpl.wait()
    cpb.wait()
    cpb.wait()

    def grp_body(grp, _):
        row0 = grp * _L
        lv = lbl_st[pl.ds(row0, _L)]
        vscr[...] = jnp.where(lv >= 0, 1.0, 0.0)
        off = row0 * 4
        for k in range(4):
            mv = plsc.load_gather(vscr, [box_sel[k]])
            bx = bbuf[pl.ds(off + k * _L, _L)]
            box_st[pl.ds(off + k * _L, _L)] = jnp.where(mv > 0.0, bx, 0.0)
        return 0

    lax.fori_loop(0, (_BPW * _QT) // _L, grp_body, 0)

    for i in range(_BPW):
        pltpu.sync_copy(box_st.at[pl.ds(i * _QT * 4, _QT * 4)],
                        boxes_out_hbm.at[b0 + i])


def _finalize_body(cref, tref, oref):
    oref[...] = jnp.maximum(jnp.sum(cref[...]) + tref[...], 1.0)


def kernel(pred_logits, pred_boxes):
    boxes2d = pred_boxes.reshape(_B, _Q * 4)
    mesh = plsc.VectorSubcoreMesh(core_axis_name="c", subcore_axis_name="s")
    labels_sc, boxes2d_sc, counts = pl.kernel(
        _sc_body,
        out_type=(
            jax.ShapeDtypeStruct((_B, _QS), jnp.int32),
            jax.ShapeDtypeStruct((_B, _QS * 4), jnp.float32),
            jax.ShapeDtypeStruct((_NW, _L), jnp.float32),
        ),
        mesh=mesh,
        compiler_params=pltpu.CompilerParams(needs_layout_passes=False),
        scratch_types=[
            pltpu.VMEM((_CHUNK, _C), jnp.float32),
            pltpu.VMEM((_CHUNK, _C), jnp.float32),
            pltpu.VMEM((_RPW * 4,), jnp.float32),
            pltpu.VMEM((_RPW,), jnp.int32),
            pltpu.VMEM((_RPW * 4,), jnp.float32),
            pltpu.VMEM((_L,), jnp.float32),
            pltpu.SemaphoreType.DMA,
            pltpu.SemaphoreType.DMA,
            pltpu.SemaphoreType.DMA,
        ],
    )(pred_logits, boxes2d)
    labels_tc, cnt_tc = pl.pallas_call(
        _tc_body,
        grid=(_B // _TB, _QT // _CHUNK),
        in_specs=[
            pl.BlockSpec((_TB, _CHUNK, _C), lambda i, j: (i, _KQ + j, 0)),
        ],
        out_specs=[
            pl.BlockSpec((_TB, _CHUNK), lambda i, j: (i, j)),
            pl.BlockSpec((1, 1), lambda i, j: (0, 0)),
        ],
        out_shape=[
            jax.ShapeDtypeStruct((_B, _QT), jnp.int32),
            jax.ShapeDtypeStruct((1, 1), jnp.float32),
        ],
    )(pred_logits)
    boxes2d_tc = pl.kernel(
        _sc2_body,
        out_type=jax.ShapeDtypeStruct((_B, _QT * 4), jnp.float32),
        mesh=mesh,
        compiler_params=pltpu.CompilerParams(needs_layout_passes=False),
        scratch_types=[
            pltpu.VMEM((_BPW * _QT,), jnp.int32),
            pltpu.VMEM((_BPW * _QT * 4,), jnp.float32),
            pltpu.VMEM((_BPW * _QT * 4,), jnp.float32),
            pltpu.VMEM((_L,), jnp.float32),
            pltpu.SemaphoreType.DMA,
            pltpu.SemaphoreType.DMA,
        ],
    )(labels_tc, boxes2d)
    num_boxes = pl.pallas_call(
        _finalize_body,
        out_shape=jax.ShapeDtypeStruct((1, 1), jnp.float32),
    )(counts, cnt_tc)[0, 0]
    labels = jnp.concatenate([labels_sc, labels_tc], axis=1)
    boxes_out = jnp.concatenate(
        [boxes2d_sc.reshape(_B, _QS, 4),
         boxes2d_tc.reshape(_B, _QT, 4)], axis=1)
    return labels, boxes_out, num_boxes
